# baseline (speedup 1.0000x reference)
import jax
import jax.numpy as jnp
from jax import lax
from jax.experimental import pallas as pl
from jax.experimental.pallas import tpu as pltpu


def _round_up(x, m):
    return (x + m - 1) // m * m


_TILE = 512
_ROT = 16


def _gather_kernel(idx_sref, tgt_sref, nrows_sref, emb_ref,
                   logits_ref, tileloss_ref, tab, lse3, *, padded):
    i = pl.program_id(0)
    j = pl.program_id(1)
    nj = pl.num_programs(1)
    tile = logits_ref.shape[0]
    v = logits_ref.shape[2]
    base = (i * nj + j) * tile
    nrows = nrows_sref[0]

    @pl.when(j == 0)
    def _build_tab():
        def _relay(c, carry):
            tab[pl.ds(c * 128, 128)] = emb_ref[pl.ds(c * 128, 128), :][:, None, :]
            return carry
        lax.fori_loop(0, v // 128, _relay, 0)

        def _lse_chunk(c, carry):
            x = emb_ref[pl.ds(c * 256, 256), :]
            mx = jnp.max(x, axis=1, keepdims=True)
            lse = mx + jnp.log(jnp.sum(jnp.exp(x - mx), axis=1, keepdims=True))
            lse3[pl.ds(c * 256, 256)] = lse[:, :, None]
            return carry
        lax.fori_loop(0, v // 256, _lse_chunk, 0)

    lane = lax.broadcasted_iota(jnp.int32, (1, 128), 1)

    def _group(g, carry):
        accs = list(carry)
        rows = []
        for u in range(_ROT):
            m = g * _ROT + u
            s = idx_sref[base + m]
            t = tgt_sref[base + m]
            rows.append(tab[s])
            c0 = pl.multiple_of((t >> 7) << 7, 128)
            chunk = tab[s, :, pl.ds(c0, 128)]
            if padded:
                valid = base + m < nrows
                msk = jnp.logical_and(lane == (t & 127), valid)
                accs[u] = accs[u] + jnp.where(msk, chunk, 0.0)
                accs[_ROT + u] = accs[_ROT + u] + jnp.where(
                    valid, lse3[s], jnp.zeros((1, 1), jnp.float32))
            else:
                accs[u] = accs[u] + jnp.where(lane == (t & 127), chunk, 0.0)
                accs[_ROT + u] = accs[_ROT + u] + lse3[s]
        for u in range(_ROT):
            logits_ref[g * _ROT + u] = rows[u]
        return tuple(accs)

    init = tuple(jnp.zeros((1, 128), jnp.float32) for _ in range(_ROT)) + \
           tuple(jnp.zeros((1, 1), jnp.float32) for _ in range(_ROT))
    accs = lax.fori_loop(0, tile // _ROT, _group, init, unroll=True)

    tl_total = jnp.sum(sum(accs[:_ROT]), axis=1, keepdims=True)
    lse_total = sum(accs[_ROT:])
    tileloss_ref[...] = (lse_total - tl_total)[None]


@jax.jit
def _bigram_train(emb, idx_flat, tgt_flat):
    bt = idx_flat.shape[0]
    v = emb.shape[1]
    bt_pad = _round_up(bt, 2 * _TILE)
    nt = bt_pad // _TILE
    nj = nt // 2

    idx_pad = jnp.zeros((bt_pad,), jnp.int32).at[:bt].set(idx_flat)
    tgt_pad = jnp.zeros((bt_pad,), jnp.int32).at[:bt].set(tgt_flat)

    grid_spec = pltpu.PrefetchScalarGridSpec(
        num_scalar_prefetch=3,
        grid=(2, nj),
        in_specs=[
            pl.BlockSpec((v, v), lambda i, j, *_: (0, 0)),
        ],
        out_specs=[
            pl.BlockSpec((_TILE, 1, v), lambda i, j, *_: (i * nj + j, 0, 0)),
            pl.BlockSpec((1, 1, 1), lambda i, j, *_: (i * nj + j, 0, 0)),
        ],
        scratch_shapes=[
            pltpu.VMEM((v, 1, v), jnp.float32),
            pltpu.VMEM((v, 1, 1), jnp.float32),
        ],
    )

    nrows = jnp.full((1,), bt, jnp.int32)
    import functools as _ft
    logits, tileloss = pl.pallas_call(
        _ft.partial(_gather_kernel, padded=(bt != bt_pad)),
        grid_spec=grid_spec,
        out_shape=(
            jax.ShapeDtypeStruct((bt_pad, 1, v), emb.dtype),
            jax.ShapeDtypeStruct((nt, 1, 1), jnp.float32),
        ),
        compiler_params=pltpu.CompilerParams(
            dimension_semantics=("parallel", "arbitrary"),
            vmem_limit_bytes=50 * 1024 * 1024,
        ),
    )(idx_pad, tgt_pad, nrows, emb)

    loss = jnp.sum(tileloss) / bt
    return logits.reshape(bt_pad, v)[:bt], loss


def kernel(emb, idx, targets):
    b, tseq = idx.shape
    v = emb.shape[1]
    idx_flat = idx.reshape(b * tseq).astype(jnp.int32)
    if targets is None:
        logits, _ = _bigram_train(emb, idx_flat,
                                  jnp.zeros((b * tseq,), jnp.int32))
        return logits.reshape(b, tseq, v), None
    tgt_flat = targets.reshape(b * tseq).astype(jnp.int32)
    logits, loss = _bigram_train(emb, idx_flat, tgt_flat)
    return logits, loss


# speedup vs baseline: 1.4291x; 1.4291x over previous
import jax
import jax.numpy as jnp
from jax import lax
from jax.experimental import pallas as pl
from jax.experimental.pallas import tpu as pltpu


def _round_up(x, m):
    return (x + m - 1) // m * m


_TILE = 512
_ROT = 16


def _gather_kernel(idx_sref, tgt_sref, nrows_sref, emb_hbm,
                   logits_ref, tileloss_ref, embs, tab, lse3, sems, *, padded):
    i = pl.program_id(0)
    j = pl.program_id(1)
    nj = pl.num_programs(1)
    tile, v = logits_ref.shape
    base = (i * nj + j) * tile
    nrows = nrows_sref[0]

    nch = 8
    ch = v // nch
    @pl.when(j == 0)
    def _build_tab():
        def _start(c, carry):
            pltpu.make_async_copy(emb_hbm.at[pl.ds(c * ch, ch), :],
                                  embs.at[pl.ds(c * ch, ch), :],
                                  sems.at[c]).start()
            return carry
        lax.fori_loop(0, nch, _start, 0)

        def _chunk(c, carry):
            pltpu.make_async_copy(emb_hbm.at[pl.ds(c * ch, ch), :],
                                  embs.at[pl.ds(c * ch, ch), :],
                                  sems.at[c]).wait()
            rs = min(128, ch)
            def _relay(r, carry2):
                tab[pl.ds(r * rs, rs)] = embs[pl.ds(r * rs, rs), :][:, None, :]
                return carry2
            lax.fori_loop(c * (ch // rs), (c + 1) * (ch // rs), _relay, 0)
            x = embs[pl.ds(c * ch, ch), :]
            mx = jnp.max(x, axis=1, keepdims=True)
            lse = mx + jnp.log(jnp.sum(jnp.exp(x - mx), axis=1, keepdims=True))
            lse3[pl.ds(c * ch, ch)] = lse[:, :, None]
            return carry
        lax.fori_loop(0, nch, _chunk, 0)

    lane = lax.broadcasted_iota(jnp.int32, (1, 128), 1)

    def _group(g, carry):
        accs = list(carry)
        rows = []
        for u in range(_ROT):
            m = g * _ROT + u
            s = idx_sref[base + m]
            t = tgt_sref[base + m]
            rows.append(tab[s])
            c0 = pl.multiple_of((t >> 7) << 7, 128)
            chunk = tab[s, :, pl.ds(c0, 128)]
            if padded:
                valid = base + m < nrows
                msk = jnp.logical_and(lane == (t & 127), valid)
                accs[u] = accs[u] + jnp.where(msk, chunk, 0.0)
                accs[_ROT + u] = accs[_ROT + u] + jnp.where(
                    valid, lse3[s], jnp.zeros((1, 1), jnp.float32))
            else:
                accs[u] = accs[u] + jnp.where(lane == (t & 127), chunk, 0.0)
                accs[_ROT + u] = accs[_ROT + u] + lse3[s]
        logits_ref[pl.ds(pl.multiple_of(g * _ROT, 8), 8), :] = jnp.concatenate(
            rows[:8], axis=0)
        logits_ref[pl.ds(pl.multiple_of(g * _ROT + 8, 8), 8), :] = jnp.concatenate(
            rows[8:], axis=0)
        return tuple(accs)

    init = tuple(jnp.zeros((1, 128), jnp.float32) for _ in range(_ROT)) + \
           tuple(jnp.zeros((1, 1), jnp.float32) for _ in range(_ROT))
    accs = lax.fori_loop(0, tile // _ROT, _group, init, unroll=True)

    tl_total = jnp.sum(sum(accs[:_ROT]), axis=1, keepdims=True)
    lse_total = sum(accs[_ROT:])
    tileloss_ref[...] = (lse_total - tl_total)[None]


@jax.jit
def _bigram_train(emb, idx_flat, tgt_flat):
    bt = idx_flat.shape[0]
    v = emb.shape[1]
    bt_pad = _round_up(bt, 2 * _TILE)
    nt = bt_pad // _TILE
    nj = nt // 2

    idx_pad = jnp.zeros((bt_pad,), jnp.int32).at[:bt].set(idx_flat)
    tgt_pad = jnp.zeros((bt_pad,), jnp.int32).at[:bt].set(tgt_flat)

    grid_spec = pltpu.PrefetchScalarGridSpec(
        num_scalar_prefetch=3,
        grid=(2, nj),
        in_specs=[
            pl.BlockSpec(memory_space=pl.ANY),
        ],
        out_specs=[
            pl.BlockSpec((_TILE, v), lambda i, j, *_: (i * nj + j, 0)),
            pl.BlockSpec((1, 1, 1), lambda i, j, *_: (i * nj + j, 0, 0)),
        ],
        scratch_shapes=[
            pltpu.VMEM((v, v), jnp.float32),
            pltpu.VMEM((v, 1, v), jnp.float32),
            pltpu.VMEM((v, 1, 1), jnp.float32),
            pltpu.SemaphoreType.DMA((8,)),
        ],
    )

    nrows = jnp.full((1,), bt, jnp.int32)
    import functools as _ft
    logits, tileloss = pl.pallas_call(
        _ft.partial(_gather_kernel, padded=(bt != bt_pad)),
        grid_spec=grid_spec,
        out_shape=(
            jax.ShapeDtypeStruct((bt_pad, v), emb.dtype),
            jax.ShapeDtypeStruct((nt, 1, 1), jnp.float32),
        ),
        compiler_params=pltpu.CompilerParams(
            dimension_semantics=("parallel", "arbitrary"),
            vmem_limit_bytes=50 * 1024 * 1024,
        ),
    )(idx_pad, tgt_pad, nrows, emb)

    loss = jnp.sum(tileloss) / bt
    return logits[:bt], loss


def kernel(emb, idx, targets):
    b, tseq = idx.shape
    v = emb.shape[1]
    idx_flat = idx.reshape(b * tseq).astype(jnp.int32)
    if targets is None:
        logits, _ = _bigram_train(emb, idx_flat,
                                  jnp.zeros((b * tseq,), jnp.int32))
        return logits.reshape(b, tseq, v), None
    tgt_flat = targets.reshape(b * tseq).astype(jnp.int32)
    logits, loss = _bigram_train(emb, idx_flat, tgt_flat)
    return logits, loss


# speedup vs baseline: 1.4711x; 1.0294x over previous
import jax
import jax.numpy as jnp
from jax import lax
from jax.experimental import pallas as pl
from jax.experimental.pallas import tpu as pltpu


def _round_up(x, m):
    return (x + m - 1) // m * m


_TILE = 512
_ROT = 8


def _gather_kernel(pk_sref, nrows_sref, emb_hbm,
                   logits_ref, tileloss_ref, embs, tab, lse3, sems, *, padded):
    i = pl.program_id(0)
    j = pl.program_id(1)
    nj = pl.num_programs(1)
    tile, v = logits_ref.shape
    base = (i * nj + j) * tile
    nrows = nrows_sref[0]

    nch = 8
    ch = v // nch
    @pl.when(j == 0)
    def _build_tab():
        def _start(c, carry):
            pltpu.make_async_copy(emb_hbm.at[pl.ds(c * ch, ch), :],
                                  embs.at[pl.ds(c * ch, ch), :],
                                  sems.at[c]).start()
            return carry
        lax.fori_loop(0, nch, _start, 0)

        def _chunk(c, carry):
            pltpu.make_async_copy(emb_hbm.at[pl.ds(c * ch, ch), :],
                                  embs.at[pl.ds(c * ch, ch), :],
                                  sems.at[c]).wait()
            rs = min(128, ch)
            def _relay(r, carry2):
                tab[pl.ds(r * rs, rs)] = embs[pl.ds(r * rs, rs), :][:, None, :]
                return carry2
            lax.fori_loop(c * (ch // rs), (c + 1) * (ch // rs), _relay, 0)
            x = embs[pl.ds(c * ch, ch), :]
            mx = jnp.max(x, axis=1, keepdims=True)
            lse = mx + jnp.log(jnp.sum(jnp.exp(x - mx), axis=1, keepdims=True))
            lse3[pl.ds(c * ch, ch)] = lse[:, :, None]
            return carry
        lax.fori_loop(0, nch, _chunk, 0)

    lane = lax.broadcasted_iota(jnp.int32, (1, 128), 1)

    def _group(g, carry):
        accs = list(carry)
        rows = []
        for u in range(_ROT):
            m = g * _ROT + u
            p = pk_sref[base + m]
            s = p >> 16
            t = p & 65535
            rows.append(tab[s])
            c0 = pl.multiple_of((t >> 7) << 7, 128)
            chunk = tab[s, :, pl.ds(c0, 128)]
            if padded:
                valid = base + m < nrows
                msk = jnp.logical_and(lane == (t & 127), valid)
                accs[u] = accs[u] + jnp.where(msk, chunk, 0.0)
                accs[_ROT + u] = accs[_ROT + u] + jnp.where(
                    valid, lse3[s], jnp.zeros((1, 1), jnp.float32))
            else:
                accs[u] = accs[u] + jnp.where(lane == (t & 127), chunk, 0.0)
                accs[_ROT + u] = accs[_ROT + u] + lse3[s]
        logits_ref[pl.ds(pl.multiple_of(g * _ROT, 8), 8), :] = jnp.concatenate(
            rows, axis=0)
        return tuple(accs)

    init = tuple(jnp.zeros((1, 128), jnp.float32) for _ in range(_ROT)) + \
           tuple(jnp.zeros((1, 1), jnp.float32) for _ in range(_ROT))
    accs = lax.fori_loop(0, tile // _ROT, _group, init, unroll=True)

    tl_total = jnp.sum(sum(accs[:_ROT]), axis=1, keepdims=True)
    lse_total = sum(accs[_ROT:])
    tileloss_ref[...] = (lse_total - tl_total)[None]


@jax.jit
def _bigram_train(emb, idx_flat, tgt_flat):
    bt = idx_flat.shape[0]
    v = emb.shape[1]
    bt_pad = _round_up(bt, 2 * _TILE)
    nt = bt_pad // _TILE
    nj = nt // 2

    packed = jnp.zeros((bt_pad,), jnp.int32).at[:bt].set(
        idx_flat * 65536 + tgt_flat)

    grid_spec = pltpu.PrefetchScalarGridSpec(
        num_scalar_prefetch=2,
        grid=(2, nj),
        in_specs=[
            pl.BlockSpec(memory_space=pl.ANY),
        ],
        out_specs=[
            pl.BlockSpec((_TILE, v), lambda i, j, *_: (i * nj + j, 0)),
            pl.BlockSpec((1, 1, 1), lambda i, j, *_: (i * nj + j, 0, 0)),
        ],
        scratch_shapes=[
            pltpu.VMEM((v, v), jnp.float32),
            pltpu.VMEM((v, 1, v), jnp.float32),
            pltpu.VMEM((v, 1, 1), jnp.float32),
            pltpu.SemaphoreType.DMA((8,)),
        ],
    )

    nrows = jnp.full((1,), bt, jnp.int32)
    import functools as _ft
    logits, tileloss = pl.pallas_call(
        _ft.partial(_gather_kernel, padded=(bt != bt_pad)),
        grid_spec=grid_spec,
        out_shape=(
            jax.ShapeDtypeStruct((bt_pad, v), emb.dtype),
            jax.ShapeDtypeStruct((nt, 1, 1), jnp.float32),
        ),
        compiler_params=pltpu.CompilerParams(
            dimension_semantics=("parallel", "arbitrary"),
            vmem_limit_bytes=50 * 1024 * 1024,
        ),
    )(packed, nrows, emb)

    loss = jnp.sum(tileloss) / bt
    return logits[:bt], loss


def kernel(emb, idx, targets):
    b, tseq = idx.shape
    v = emb.shape[1]
    idx_flat = idx.reshape(b * tseq).astype(jnp.int32)
    if targets is None:
        logits, _ = _bigram_train(emb, idx_flat,
                                  jnp.zeros((b * tseq,), jnp.int32))
        return logits.reshape(b, tseq, v), None
    tgt_flat = targets.reshape(b * tseq).astype(jnp.int32)
    logits, loss = _bigram_train(emb, idx_flat, tgt_flat)
    return logits, loss
